# baseline (speedup 1.0000x reference)
import functools

import jax
import jax.numpy as jnp
from jax import lax
from jax.experimental import pallas as pl
from jax.experimental.pallas import tpu as pltpu
from jax.experimental.pallas import tpu_sc as plsc

_N = 100000
_B = 128
_NW = 32
_RPW = _N // _NW
_CH = 125
_NCHK = _RPW // _CH
_CHF = _CH * _B
_NK = _B // 16


def _wid():
    return lax.axis_index("s") * 2 + lax.axis_index("c")


def _stats_body(x_hbm, m_hbm, s_hbm, buf, stat):
    w = _wid()
    base = w * _RPW * _B
    minf = jnp.full((16,), -jnp.inf, jnp.float32)
    zero = jnp.zeros((16,), jnp.float32)

    def chunk_loop(t, carry):
        pltpu.sync_copy(x_hbm.at[pl.ds(base + t * _CHF, _CHF)], buf)
        ms, ss = carry[:_NK], carry[_NK:]

        def maxr(r, cms):
            return tuple(
                jnp.maximum(cms[k], buf[pl.ds(r * _B + 16 * k, 16)])
                for k in range(_NK)
            )

        cm = lax.fori_loop(0, _CH, maxr, (minf,) * _NK)
        nm = tuple(jnp.maximum(ms[k], cm[k]) for k in range(_NK))
        ns0 = tuple(ss[k] * jnp.exp(ms[k] - nm[k]) for k in range(_NK))

        def expr(r, sacc):
            return tuple(
                sacc[k] + jnp.exp(buf[pl.ds(r * _B + 16 * k, 16)] - nm[k])
                for k in range(_NK)
            )

        ns = lax.fori_loop(0, _CH, expr, ns0)
        return nm + ns

    carry = lax.fori_loop(
        0, _NCHK, chunk_loop, (minf,) * _NK + (zero,) * _NK
    )
    for k in range(_NK):
        stat[pl.ds(16 * k, 16)] = carry[k]
        stat[pl.ds(_B + 16 * k, 16)] = carry[_NK + k]
    pltpu.sync_copy(stat.at[pl.ds(0, _B)], m_hbm.at[pl.ds(w * _B, _B)])
    pltpu.sync_copy(stat.at[pl.ds(_B, _B)], s_hbm.at[pl.ds(w * _B, _B)])


def _scale_body(x_hbm, g_hbm, o_hbm, buf, stat):
    w = _wid()
    base = w * _RPW * _B
    pltpu.sync_copy(g_hbm, stat)
    gm = tuple(stat[pl.ds(16 * k, 16)] for k in range(_NK))
    gi = tuple(stat[pl.ds(_B + 16 * k, 16)] for k in range(_NK))

    def chunk_loop(t, carry):
        pltpu.sync_copy(x_hbm.at[pl.ds(base + t * _CHF, _CHF)], buf)

        def rowb(r, c):
            for k in range(_NK):
                sl = pl.ds(r * _B + 16 * k, 16)
                buf[sl] = jnp.exp(buf[sl] - gm[k]) * gi[k]
            return c

        lax.fori_loop(0, _CH, rowb, 0)
        pltpu.sync_copy(buf, o_hbm.at[pl.ds(base + t * _CHF, _CHF)])
        return carry

    lax.fori_loop(0, _NCHK, chunk_loop, 0)


def kernel(logits):
    b, n = logits.shape
    xf = logits.T.reshape((n * b,))
    mesh = plsc.VectorSubcoreMesh(core_axis_name="c", subcore_axis_name="s")

    stats_fn = pl.kernel(
        _stats_body,
        out_type=(
            jax.ShapeDtypeStruct((_NW * _B,), jnp.float32),
            jax.ShapeDtypeStruct((_NW * _B,), jnp.float32),
        ),
        mesh=mesh,
        scratch_types=[
            pltpu.VMEM((_CHF,), jnp.float32),
            pltpu.VMEM((2 * _B,), jnp.float32),
        ],
    )
    m_w, s_w = stats_fn(xf)
    m_w = m_w.reshape((_NW, _B))
    s_w = s_w.reshape((_NW, _B))
    m_g = jnp.max(m_w, axis=0, keepdims=True)
    s_g = jnp.sum(s_w * jnp.exp(m_w - m_g), axis=0)
    g = jnp.concatenate([m_g[0], 1.0 / s_g])

    scale_fn = pl.kernel(
        _scale_body,
        out_type=jax.ShapeDtypeStruct((n * b,), jnp.float32),
        mesh=mesh,
        scratch_types=[
            pltpu.VMEM((_CHF,), jnp.float32),
            pltpu.VMEM((2 * _B,), jnp.float32),
        ],
    )
    of = scale_fn(xf, g)
    return of.reshape((n, b)).T


# speedup vs baseline: 4.0181x; 4.0181x over previous
import functools

import jax
import jax.numpy as jnp
from jax import lax
from jax.experimental import pallas as pl
from jax.experimental.pallas import tpu as pltpu

_CHUNK = 5000


def _softmax_t(x_hbm, o_hbm, xbuf, stat, in_sem, out_sem, *, n, b):
    nch = n // _CHUNK

    def in_copy(c):
        sl = pl.ds(c * _CHUNK, _CHUNK)
        return pltpu.make_async_copy(x_hbm.at[sl], xbuf.at[sl], in_sem.at[c])

    def out_copy(c):
        sl = pl.ds(c * _CHUNK, _CHUNK)
        return pltpu.make_async_copy(xbuf.at[sl], o_hbm.at[sl], out_sem.at[c])

    for c in range(nch):
        in_copy(c).start()

    def exp_body(c, m):
        in_copy(c).wait()
        sl = pl.ds(c * _CHUNK, _CHUNK)
        x = xbuf[sl, :]
        cm = jnp.max(x, axis=0, keepdims=True)
        e = jnp.exp(x - cm)
        xbuf[sl, :] = e
        cs = jnp.sum(e, axis=0, keepdims=True)
        stat[pl.ds(8 * c, 2), :] = jnp.concatenate([cm, cs], axis=0)
        return jnp.maximum(m, cm)

    m = lax.fori_loop(
        0, nch, exp_body, jnp.full((1, b), -jnp.inf, jnp.float32)
    )

    def sum_body(c, s):
        st = stat[pl.ds(8 * c, 2), :]
        return s + st[1:2, :] * jnp.exp(st[0:1, :] - m)

    s = lax.fori_loop(0, nch, sum_body, jnp.zeros((1, b), jnp.float32))
    inv = 1.0 / s

    def scale_body(c, carry):
        sl = pl.ds(c * _CHUNK, _CHUNK)
        f = jnp.exp(stat[pl.ds(8 * c, 1), :] - m) * inv
        xbuf[sl, :] = xbuf[sl, :] * f
        out_copy(c).start()
        return carry

    lax.fori_loop(0, nch, scale_body, 0)

    def drain_body(c, carry):
        out_copy(c).wait()
        return carry

    lax.fori_loop(0, nch, drain_body, 0)


def kernel(logits):
    b, n = logits.shape
    xt = logits.T
    nch = n // _CHUNK
    out_t = pl.pallas_call(
        functools.partial(_softmax_t, n=n, b=b),
        in_specs=[pl.BlockSpec(memory_space=pl.ANY)],
        out_specs=pl.BlockSpec(memory_space=pl.ANY),
        out_shape=jax.ShapeDtypeStruct((n, b), jnp.float32),
        scratch_shapes=[
            pltpu.VMEM((n, b), jnp.float32),
            pltpu.VMEM((8 * nch, b), jnp.float32),
            pltpu.SemaphoreType.DMA((nch,)),
            pltpu.SemaphoreType.DMA((nch,)),
        ],
    )(xt)
    return out_t.T
